# baseline (speedup 1.0000x reference)
import jax
import jax.numpy as jnp
from jax.experimental import pallas as pl

_HIDDEN = 32
_INTER = 16
_EXPERTS = 4
_BLOCK = 16384

_RHS_T = (((1,), (1,)), ((), ()))
_NN = (((1,), (0,)), ((), ()))


def _moe_body(x_ref, wr_ref, wgu_ref, wd_ref, sel_ref, o_ref):
    x = x_ref[...]
    f32 = jnp.float32

    l4 = jax.lax.dot_general(wr_ref[...], x, _RHS_T, preferred_element_type=f32)

    l1 = jnp.max(l4, axis=0, keepdims=True)
    eidx4 = jax.lax.broadcasted_iota(jnp.int32, l4.shape, 0)
    i1 = jnp.min(jnp.where(l4 == l1, eidx4, _EXPERTS), axis=0, keepdims=True)
    lm = jnp.where(eidx4 == i1, -jnp.inf, l4)
    l2 = jnp.max(lm, axis=0, keepdims=True)
    i2 = jnp.min(jnp.where(lm == l2, eidx4, _EXPERTS), axis=0, keepdims=True)

    e21 = jnp.exp(l2 - l1)
    s = e21 / (1.0 + e21)
    rw4 = jnp.where(eidx4 == i1, 1.0 - s, jnp.where(eidx4 == i2, s, 0.0))

    rw = jax.lax.dot_general(sel_ref[...], rw4, _NN, preferred_element_type=f32)

    gu = jax.lax.dot_general(wgu_ref[...], x, _RHS_T, preferred_element_type=f32)
    g = gu[: _EXPERTS * _INTER, :]
    u = gu[_EXPERTS * _INTER :, :]

    h = g * (1.0 / (1.0 + jnp.exp(-g))) * u * rw
    o_ref[...] = jax.lax.dot_general(h, wd_ref[...], (((0,), (1,)), ((), ())),
                                    preferred_element_type=f32)


@jax.jit
def kernel(hidden_states, router_weight, gate_up_proj, down_proj):
    batch, seq, hidden = hidden_states.shape
    n_tokens = batch * seq
    x = hidden_states.reshape(n_tokens, hidden)

    wr = router_weight.astype(jnp.float32)
    gate_w = gate_up_proj[:, :_INTER, :].reshape(_EXPERTS * _INTER, hidden)
    up_w = gate_up_proj[:, _INTER:, :].reshape(_EXPERTS * _INTER, hidden)
    wgu = jnp.concatenate([gate_w, up_w], axis=0)
    wd = jnp.transpose(down_proj, (1, 0, 2)).reshape(hidden, _EXPERTS * _INTER)
    sel = jnp.repeat(jnp.eye(_EXPERTS, dtype=jnp.float32), _INTER, axis=0)

    grid = (n_tokens // _BLOCK,)
    out = pl.pallas_call(
        _moe_body,
        grid=grid,
        in_specs=[
            pl.BlockSpec((_BLOCK, hidden), lambda i: (i, 0)),
            pl.BlockSpec(wr.shape, lambda i: (0, 0)),
            pl.BlockSpec(wgu.shape, lambda i: (0, 0)),
            pl.BlockSpec(wd.shape, lambda i: (0, 0)),
            pl.BlockSpec(sel.shape, lambda i: (0, 0)),
        ],
        out_specs=pl.BlockSpec((_BLOCK, hidden), lambda i: (i, 0)),
        out_shape=jax.ShapeDtypeStruct((n_tokens, hidden), jnp.float32),
    )(x, wr, wgu, wd, sel)
    return out.reshape(batch, seq, hidden)


# speedup vs baseline: 1.0328x; 1.0328x over previous
import jax
import jax.numpy as jnp
from jax.experimental import pallas as pl

_HIDDEN = 32
_INTER = 16
_EXPERTS = 4
_BLOCK = 8192

_RHS_T = (((1,), (1,)), ((), ()))
_NN = (((1,), (0,)), ((), ()))


def _moe_body(x_ref, wr_ref, wgu_ref, wd_ref, sel_ref, o_ref):
    x = x_ref[...]
    f32 = jnp.float32

    l4 = jax.lax.dot_general(wr_ref[...], x, _RHS_T, preferred_element_type=f32)

    l1 = jnp.max(l4, axis=0, keepdims=True)
    eidx4 = jax.lax.broadcasted_iota(jnp.int32, l4.shape, 0)
    i1 = jnp.min(jnp.where(l4 == l1, eidx4, _EXPERTS), axis=0, keepdims=True)
    lm = jnp.where(eidx4 == i1, -jnp.inf, l4)
    l2 = jnp.max(lm, axis=0, keepdims=True)
    i2 = jnp.min(jnp.where(lm == l2, eidx4, _EXPERTS), axis=0, keepdims=True)

    e21 = jnp.exp(l2 - l1)
    s = e21 / (1.0 + e21)
    rw4 = jnp.where(eidx4 == i1, 1.0 - s, jnp.where(eidx4 == i2, s, 0.0))

    rw = jnp.concatenate(
        [jnp.broadcast_to(rw4[e:e + 1], (_INTER, rw4.shape[1]))
         for e in range(_EXPERTS)], axis=0)

    gu = jax.lax.dot_general(wgu_ref[...], x, _RHS_T, preferred_element_type=f32)
    g = gu[: _EXPERTS * _INTER, :]
    u = gu[_EXPERTS * _INTER :, :]

    h = jax.nn.silu(g) * u * rw
    o_ref[...] = jax.lax.dot_general(h.astype(jnp.bfloat16), wd_ref[...],
                                    (((0,), (1,)), ((), ())),
                                    preferred_element_type=f32)


@jax.jit
def kernel(hidden_states, router_weight, gate_up_proj, down_proj):
    batch, seq, hidden = hidden_states.shape
    n_tokens = batch * seq
    x = hidden_states.reshape(n_tokens, hidden)

    wr = router_weight.astype(jnp.float32)
    gate_w = gate_up_proj[:, :_INTER, :].reshape(_EXPERTS * _INTER, hidden)
    up_w = gate_up_proj[:, _INTER:, :].reshape(_EXPERTS * _INTER, hidden)
    wgu = jnp.concatenate([gate_w, up_w], axis=0)
    wd = jnp.transpose(down_proj, (1, 0, 2)).reshape(hidden, _EXPERTS * _INTER)
    wd = wd.astype(jnp.bfloat16)
    sel = jnp.repeat(jnp.eye(_EXPERTS, dtype=jnp.float32), _INTER, axis=0)

    grid = (n_tokens // _BLOCK,)
    out = pl.pallas_call(
        _moe_body,
        grid=grid,
        in_specs=[
            pl.BlockSpec((_BLOCK, hidden), lambda i: (i, 0)),
            pl.BlockSpec(wr.shape, lambda i: (0, 0)),
            pl.BlockSpec(wgu.shape, lambda i: (0, 0)),
            pl.BlockSpec(wd.shape, lambda i: (0, 0)),
            pl.BlockSpec(sel.shape, lambda i: (0, 0)),
        ],
        out_specs=pl.BlockSpec((_BLOCK, hidden), lambda i: (i, 0)),
        out_shape=jax.ShapeDtypeStruct((n_tokens, hidden), jnp.float32),
    )(x, wr, wgu, wd, sel)
    return out.reshape(batch, seq, hidden)


# speedup vs baseline: 1.0538x; 1.0203x over previous
import jax
import jax.numpy as jnp
from jax.experimental import pallas as pl

_HIDDEN = 32
_INTER = 16
_EXPERTS = 4
_BLOCK = 8192

_RHS_T = (((1,), (1,)), ((), ()))
_NN = (((1,), (0,)), ((), ()))


def _moe_body(x_ref, wr_ref, wgu_ref, wd_ref, o_ref):
    x = x_ref[...]
    f32 = jnp.float32

    l4 = jax.lax.dot_general(wr_ref[...], x, _RHS_T, preferred_element_type=f32)

    l1 = jnp.max(l4, axis=0, keepdims=True)
    eidx4 = jax.lax.broadcasted_iota(jnp.int32, l4.shape, 0)
    i1 = jnp.min(jnp.where(l4 == l1, eidx4, _EXPERTS), axis=0, keepdims=True)
    lm = jnp.where(eidx4 == i1, -jnp.inf, l4)
    l2 = jnp.max(lm, axis=0, keepdims=True)
    i2 = jnp.min(jnp.where(lm == l2, eidx4, _EXPERTS), axis=0, keepdims=True)

    e21 = jnp.exp(l2 - l1)
    s = e21 / (1.0 + e21)
    rw4 = jnp.where(eidx4 == i1, 1.0 - s, jnp.where(eidx4 == i2, s, 0.0))

    rw = jnp.concatenate(
        [jnp.broadcast_to(rw4[e:e + 1], (_INTER, rw4.shape[1]))
         for e in range(_EXPERTS)], axis=0)

    gu = jax.lax.dot_general(wgu_ref[...], x, _RHS_T, preferred_element_type=f32)
    g = gu[: _EXPERTS * _INTER, :]
    u = gu[_EXPERTS * _INTER :, :]

    h = jax.nn.silu(g) * u * rw
    o_ref[...] = jax.lax.dot_general(h.astype(jnp.bfloat16), wd_ref[...],
                                    (((0,), (1,)), ((), ())),
                                    preferred_element_type=f32)


@jax.jit
def kernel(hidden_states, router_weight, gate_up_proj, down_proj):
    batch, seq, hidden = hidden_states.shape
    n_tokens = batch * seq
    x = hidden_states.reshape(n_tokens, hidden)

    wr = router_weight.astype(jnp.float32)
    gate_w = gate_up_proj[:, :_INTER, :].reshape(_EXPERTS * _INTER, hidden)
    up_w = gate_up_proj[:, _INTER:, :].reshape(_EXPERTS * _INTER, hidden)
    wgu = jnp.concatenate([gate_w, up_w], axis=0)
    wd = jnp.transpose(down_proj, (1, 0, 2)).reshape(hidden, _EXPERTS * _INTER)
    wd = wd.astype(jnp.bfloat16)
    grid = (n_tokens // _BLOCK,)
    out = pl.pallas_call(
        _moe_body,
        grid=grid,
        in_specs=[
            pl.BlockSpec((_BLOCK, hidden), lambda i: (i, 0)),
            pl.BlockSpec(wr.shape, lambda i: (0, 0)),
            pl.BlockSpec(wgu.shape, lambda i: (0, 0)),
            pl.BlockSpec(wd.shape, lambda i: (0, 0)),
        ],
        out_specs=pl.BlockSpec((_BLOCK, hidden), lambda i: (i, 0)),
        out_shape=jax.ShapeDtypeStruct((n_tokens, hidden), jnp.float32),
    )(x, wr, wgu, wd)
    return out.reshape(batch, seq, hidden)
